# baseline (speedup 1.0000x reference)
import jax
import jax.numpy as jnp
import numpy as np
from jax.experimental import pallas as pl
from jax.experimental.pallas import tpu as pltpu

NUM_BUCKETS = 32
NUM_HEADS = 16
Q = 2048
K = 2048
ROWS_PER_BLOCK = 128
TBL_W = 4096
OFF = 2040


def _pbe_kernel(embT_ref, out_ref, tbl_ref):
    a = pl.program_id(1)

    @pl.when(a == 0)
    def _build_table():
        s = jax.lax.broadcasted_iota(jnp.int32, (8, TBL_W), 0)
        x = jax.lax.broadcasted_iota(jnp.int32, (8, TBL_W), 1)
        d = x - OFF - s
        n = -d
        ret = jnp.where(n < 0, 16, 0)
        n = jnp.abs(n)
        is_small = n < 8
        n_safe = jnp.maximum(n, 1).astype(jnp.float32)
        val = 8 + (jnp.log(n_safe / 8) / np.log(128 / 8) * 8).astype(jnp.int32)
        val = jnp.minimum(val, 15)
        b = ret + jnp.where(is_small, n, val)
        acc = jnp.zeros((8, TBL_W), jnp.float32)
        for j in range(NUM_BUCKETS):
            acc = acc + jnp.where(b == j, embT_ref[0, 0, j], 0.0)
        tbl_ref[...] = acc

    @pl.when(a > 0)
    def _advance_table():
        tbl_ref[...] = pltpu.roll(tbl_ref[...], 128, 1)

    for g in range(ROWS_PER_BLOCK // 8):
        start = OFF - 8 * g
        out_ref[0, 8 * g:8 * g + 8, :] = tbl_ref[:, start:start + K]


def kernel(embedding, query_length, key_length):
    del query_length, key_length
    embT = embedding.T.reshape(NUM_HEADS, 1, NUM_BUCKETS)
    out = pl.pallas_call(
        _pbe_kernel,
        grid=(NUM_HEADS, Q // ROWS_PER_BLOCK),
        in_specs=[pl.BlockSpec((1, 1, NUM_BUCKETS), lambda h, a: (h, 0, 0))],
        out_specs=pl.BlockSpec((1, ROWS_PER_BLOCK, K), lambda h, a: (h, a, 0)),
        out_shape=jax.ShapeDtypeStruct((NUM_HEADS, Q, K), jnp.float32),
        scratch_shapes=[pltpu.VMEM((8, TBL_W), jnp.float32)],
        compiler_params=pltpu.CompilerParams(
            dimension_semantics=("parallel", "arbitrary")),
    )(embT)
    return out[None]


# speedup vs baseline: 1.3459x; 1.3459x over previous
import jax
import jax.numpy as jnp
import numpy as np
from jax.experimental import pallas as pl
from jax.experimental.pallas import tpu as pltpu

NUM_BUCKETS = 32
NUM_HEADS = 16
Q = 2048
K = 2048
ROWS_PER_BLOCK = 256
TBL_W = 4096
OFF = 2040


def _pbe_kernel(embT_ref, out_ref, tbl_ref):
    a = pl.program_id(1)

    @pl.when(a == 0)
    def _build_table():
        s = jax.lax.broadcasted_iota(jnp.int32, (8, TBL_W), 0)
        x = jax.lax.broadcasted_iota(jnp.int32, (8, TBL_W), 1)
        d = x - OFF - s
        n = -d
        ret = jnp.where(n < 0, 16, 0)
        n = jnp.abs(n)
        is_small = n < 8
        n_safe = jnp.maximum(n, 1).astype(jnp.float32)
        val = 8 + (jnp.log(n_safe / 8) / np.log(128 / 8) * 8).astype(jnp.int32)
        val = jnp.minimum(val, 15)
        b = ret + jnp.where(is_small, n, val)
        acc = jnp.zeros((8, TBL_W), jnp.float32)
        for j in range(NUM_BUCKETS):
            acc = acc + jnp.where(b == j, embT_ref[0, 0, j], 0.0)
        tbl_ref[...] = acc

    @pl.when(a > 0)
    def _advance_table():
        tbl_ref[...] = pltpu.roll(tbl_ref[...], ROWS_PER_BLOCK, 1)

    for g in range(ROWS_PER_BLOCK // 8):
        start = OFF - 8 * g
        out_ref[0, 8 * g:8 * g + 8, :] = tbl_ref[:, start:start + K]


def kernel(embedding, query_length, key_length):
    del query_length, key_length
    embT = embedding.T.reshape(NUM_HEADS, 1, NUM_BUCKETS)
    out = pl.pallas_call(
        _pbe_kernel,
        grid=(NUM_HEADS, Q // ROWS_PER_BLOCK),
        in_specs=[pl.BlockSpec((1, 1, NUM_BUCKETS), lambda h, a: (h, 0, 0))],
        out_specs=pl.BlockSpec((1, ROWS_PER_BLOCK, K), lambda h, a: (h, a, 0)),
        out_shape=jax.ShapeDtypeStruct((NUM_HEADS, Q, K), jnp.float32),
        scratch_shapes=[pltpu.VMEM((8, TBL_W), jnp.float32)],
        compiler_params=pltpu.CompilerParams(
            dimension_semantics=("parallel", "arbitrary")),
    )(embT)
    return out[None]


# speedup vs baseline: 1.6173x; 1.2016x over previous
import jax
import jax.numpy as jnp
import numpy as np
from jax.experimental import pallas as pl
from jax.experimental.pallas import tpu as pltpu

NUM_BUCKETS = 32
NUM_HEADS = 16
Q = 2048
K = 2048
ROWS_PER_BLOCK = 512
TBL_W = 4096
OFF = 2040


def _pbe_kernel(embT_ref, out_ref, tbl_ref):
    a = pl.program_id(1)

    @pl.when(a == 0)
    def _build_table():
        s = jax.lax.broadcasted_iota(jnp.int32, (8, TBL_W), 0)
        x = jax.lax.broadcasted_iota(jnp.int32, (8, TBL_W), 1)
        d = x - OFF - s
        n = -d
        ret = jnp.where(n < 0, 16, 0)
        n = jnp.abs(n)
        is_small = n < 8
        n_safe = jnp.maximum(n, 1).astype(jnp.float32)
        val = 8 + (jnp.log(n_safe / 8) / np.log(128 / 8) * 8).astype(jnp.int32)
        val = jnp.minimum(val, 15)
        b = ret + jnp.where(is_small, n, val)
        acc = jnp.zeros((8, TBL_W), jnp.float32)
        for j in range(NUM_BUCKETS):
            acc = acc + jnp.where(b == j, embT_ref[0, 0, j], 0.0)
        tbl_ref[...] = acc

    @pl.when(a > 0)
    def _advance_table():
        tbl_ref[...] = pltpu.roll(tbl_ref[...], ROWS_PER_BLOCK, 1)

    for g in range(ROWS_PER_BLOCK // 8):
        start = OFF - 8 * g
        out_ref[0, 8 * g:8 * g + 8, :] = tbl_ref[:, start:start + K]


def kernel(embedding, query_length, key_length):
    del query_length, key_length
    embT = embedding.T.reshape(NUM_HEADS, 1, NUM_BUCKETS)
    out = pl.pallas_call(
        _pbe_kernel,
        grid=(NUM_HEADS, Q // ROWS_PER_BLOCK),
        in_specs=[pl.BlockSpec((1, 1, NUM_BUCKETS), lambda h, a: (h, 0, 0))],
        out_specs=pl.BlockSpec((1, ROWS_PER_BLOCK, K), lambda h, a: (h, a, 0)),
        out_shape=jax.ShapeDtypeStruct((NUM_HEADS, Q, K), jnp.float32),
        scratch_shapes=[pltpu.VMEM((8, TBL_W), jnp.float32)],
        compiler_params=pltpu.CompilerParams(
            dimension_semantics=("parallel", "arbitrary")),
    )(embT)
    return out[None]


# speedup vs baseline: 1.8085x; 1.1182x over previous
import jax
import jax.numpy as jnp
import numpy as np
from jax.experimental import pallas as pl
from jax.experimental.pallas import tpu as pltpu

NUM_BUCKETS = 32
NUM_HEADS = 16
Q = 2048
K = 2048
ROWS_PER_BLOCK = 1024
TBL_W = 4096
OFF = 2040


def _pbe_kernel(embT_ref, out_ref, tbl_ref):
    a = pl.program_id(1)

    @pl.when(a == 0)
    def _build_table():
        s = jax.lax.broadcasted_iota(jnp.int32, (8, TBL_W), 0)
        x = jax.lax.broadcasted_iota(jnp.int32, (8, TBL_W), 1)
        d = x - OFF - s
        n = -d
        ret = jnp.where(n < 0, 16, 0)
        n = jnp.abs(n)
        is_small = n < 8
        n_safe = jnp.maximum(n, 1).astype(jnp.float32)
        val = 8 + (jnp.log(n_safe / 8) / np.log(128 / 8) * 8).astype(jnp.int32)
        val = jnp.minimum(val, 15)
        b = ret + jnp.where(is_small, n, val)
        acc = jnp.zeros((8, TBL_W), jnp.float32)
        for j in range(NUM_BUCKETS):
            acc = acc + jnp.where(b == j, embT_ref[0, 0, j], 0.0)
        tbl_ref[...] = acc

    @pl.when(a > 0)
    def _advance_table():
        tbl_ref[...] = pltpu.roll(tbl_ref[...], ROWS_PER_BLOCK, 1)

    for g in range(ROWS_PER_BLOCK // 8):
        start = OFF - 8 * g
        out_ref[0, 8 * g:8 * g + 8, :] = tbl_ref[:, start:start + K]


def kernel(embedding, query_length, key_length):
    del query_length, key_length
    embT = embedding.T.reshape(NUM_HEADS, 1, NUM_BUCKETS)
    out = pl.pallas_call(
        _pbe_kernel,
        grid=(NUM_HEADS, Q // ROWS_PER_BLOCK),
        in_specs=[pl.BlockSpec((1, 1, NUM_BUCKETS), lambda h, a: (h, 0, 0))],
        out_specs=pl.BlockSpec((1, ROWS_PER_BLOCK, K), lambda h, a: (h, a, 0)),
        out_shape=jax.ShapeDtypeStruct((NUM_HEADS, Q, K), jnp.float32),
        scratch_shapes=[pltpu.VMEM((8, TBL_W), jnp.float32)],
        compiler_params=pltpu.CompilerParams(
            dimension_semantics=("parallel", "arbitrary")),
    )(embT)
    return out[None]


# speedup vs baseline: 2.0050x; 1.1087x over previous
import jax
import jax.numpy as jnp
import numpy as np
from jax.experimental import pallas as pl
from jax.experimental.pallas import tpu as pltpu

NUM_BUCKETS = 32
NUM_HEADS = 16
Q = 2048
K = 2048
ROWS_PER_BLOCK = 2048
TBL_W = 4096
OFF = 2040


def _pbe_kernel(embT_ref, out_ref, tbl_ref):
    a = pl.program_id(1)

    @pl.when(a == 0)
    def _build_table():
        s = jax.lax.broadcasted_iota(jnp.int32, (8, TBL_W), 0)
        x = jax.lax.broadcasted_iota(jnp.int32, (8, TBL_W), 1)
        d = x - OFF - s
        n = -d
        ret = jnp.where(n < 0, 16, 0)
        n = jnp.abs(n)
        is_small = n < 8
        n_safe = jnp.maximum(n, 1).astype(jnp.float32)
        val = 8 + (jnp.log(n_safe / 8) / np.log(128 / 8) * 8).astype(jnp.int32)
        val = jnp.minimum(val, 15)
        b = ret + jnp.where(is_small, n, val)
        acc = jnp.zeros((8, TBL_W), jnp.float32)
        for j in range(NUM_BUCKETS):
            acc = acc + jnp.where(b == j, embT_ref[0, 0, j], 0.0)
        tbl_ref[...] = acc

    @pl.when(a > 0)
    def _advance_table():
        tbl_ref[...] = pltpu.roll(tbl_ref[...], ROWS_PER_BLOCK, 1)

    for g in range(ROWS_PER_BLOCK // 8):
        start = OFF - 8 * g
        out_ref[0, 8 * g:8 * g + 8, :] = tbl_ref[:, start:start + K]


def kernel(embedding, query_length, key_length):
    del query_length, key_length
    embT = embedding.T.reshape(NUM_HEADS, 1, NUM_BUCKETS)
    out = pl.pallas_call(
        _pbe_kernel,
        grid=(NUM_HEADS, Q // ROWS_PER_BLOCK),
        in_specs=[pl.BlockSpec((1, 1, NUM_BUCKETS), lambda h, a: (h, 0, 0))],
        out_specs=pl.BlockSpec((1, ROWS_PER_BLOCK, K), lambda h, a: (h, a, 0)),
        out_shape=jax.ShapeDtypeStruct((NUM_HEADS, Q, K), jnp.float32),
        scratch_shapes=[pltpu.VMEM((8, TBL_W), jnp.float32)],
        compiler_params=pltpu.CompilerParams(
            dimension_semantics=("parallel", "arbitrary")),
    )(embT)
    return out[None]
